# baseline (speedup 1.0000x reference)
import functools

import jax
import jax.numpy as jnp
from jax import lax
from jax.experimental import pallas as pl
from jax.experimental.pallas import tpu as pltpu
from jax.experimental.pallas import tpu_sc as plsc

N1 = 12000
N2 = 10000
E1 = 384000
E2 = 320000
D0 = 64
D1 = 128
D2 = 64
NLEFT = 64

CK = 128
NTILES = 32
E1P = 385024
E2P = 327680
NC1 = E1P // (NTILES * CK)
NC2 = E2P // (NTILES * CK)
CKR = 64
NBR = 4
NCR = E2P // (NTILES * CKR)
ACC_G1 = 12032
ACC_G2 = 10112
WROW = 128

_f32 = jnp.float32


def _sc_mesh():
    return plsc.VectorSubcoreMesh(core_axis_name="c", subcore_axis_name="s")


def _make_concept_scatter():
    rps = ACC_G1 // 16

    def body(tbl, src, dst, zeros, out, sidx, didx, rows, acc, sem0, sem1):
        c = lax.axis_index("c")
        s = lax.axis_index("s")
        wid = s * 2 + c
        sems = (sem0, sem1)
        pltpu.sync_copy(zeros.at[pl.ds(s * rps, rps)], acc.at[pl.ds(s * rps, rps)])

        def load(i, b):
            base = (wid * NC1 + i) * CK
            pltpu.sync_copy(src.at[pl.ds(base, CK)], sidx.at[b])
            pltpu.sync_copy(dst.at[pl.ds(base, CK)], didx.at[b])

        load(0, 0)
        pltpu.async_copy(tbl.at[sidx.at[0]], rows.at[0], sems[0])
        plsc.subcore_barrier()

        def pair(k, carry):
            i0 = k * 2
            for b in range(2):
                i = i0 + b
                nb = 1 - b

                @pl.when(i + 1 < NC1)
                def _():
                    load(i + 1, nb)
                    pltpu.async_copy(tbl.at[sidx.at[nb]], rows.at[nb], sems[nb])

                pltpu.make_async_copy(tbl.at[sidx.at[b]], rows.at[b],
                                      sems[b]).wait()
                pltpu.sync_copy(rows.at[b], acc.at[didx.at[b]], add=True)
            return carry

        lax.fori_loop(0, NC1 // 2, pair, 0)
        plsc.subcore_barrier()
        pltpu.sync_copy(acc.at[pl.ds(s * rps, rps)], out.at[c, pl.ds(s * rps, rps)])

    return pl.kernel(
        body,
        out_type=jax.ShapeDtypeStruct((2, ACC_G1, WROW), _f32),
        mesh=_sc_mesh(),
        scratch_types=[
            pltpu.VMEM((2, CK), jnp.int32),
            pltpu.VMEM((2, CK), jnp.int32),
            pltpu.VMEM((2, CK, WROW), _f32),
            pltpu.VMEM_SHARED((ACC_G1, WROW), _f32),
            pltpu.SemaphoreType.DMA,
            pltpu.SemaphoreType.DMA,
        ],
    )


def _make_rgcn_scatter(grab_ty):
    rps = ACC_G2 // 16

    def body(*refs):
        it = iter(refs)
        tbl = next(it); src = next(it); et = next(it); dst = next(it)
        zeros = next(it)
        if grab_ty:
            xg1 = next(it); lc = next(it)
        out = next(it)
        if grab_ty:
            ty_out = next(it)
        sidx = next(it); eidx = next(it); gidx = next(it); didx = next(it)
        rows = next(it)
        acc = next(it)
        if grab_ty:
            lidx = next(it); lrows = next(it)
        sems = tuple(it)

        c = lax.axis_index("c")
        s = lax.axis_index("s")
        wid = s * 2 + c
        pltpu.sync_copy(zeros.at[pl.ds(s * rps, rps)], acc.at[pl.ds(s * rps, rps)])
        if grab_ty:
            @pl.when(wid == 0)
            def _():
                pltpu.sync_copy(lc, lidx)
                pltpu.async_copy(xg1.at[lidx], lrows, sems[0]).wait()
                pltpu.sync_copy(lrows, ty_out)

        def fire(i, b):
            base = (wid * NCR + i) * CKR
            pltpu.sync_copy(src.at[pl.ds(base, CKR)], sidx.at[b])
            pltpu.sync_copy(et.at[pl.ds(base, CKR)], eidx.at[b])
            pltpu.sync_copy(dst.at[pl.ds(base, CKR)], didx.at[b])
            for j in range(CKR // 16):
                sl = pl.ds(j * 16, 16)
                gidx[b, sl] = eidx[b, sl] * N2 + sidx[b, sl]
            pltpu.async_copy(tbl.at[gidx.at[b]], rows.at[b], sems[b])

        for b in range(NBR - 1):
            fire(b, b)
        plsc.subcore_barrier()

        def ring(k, carry):
            i0 = k * NBR
            for b in range(NBR):
                i = i0 + b
                fb = (b + NBR - 1) % NBR

                @pl.when(i + NBR - 1 < NCR)
                def _():
                    fire(i + NBR - 1, fb)

                pltpu.make_async_copy(tbl.at[gidx.at[b]], rows.at[b],
                                      sems[b]).wait()
                pltpu.sync_copy(rows.at[b], acc.at[didx.at[b]], add=True)
            return carry

        lax.fori_loop(0, NCR // NBR, ring, 0)
        plsc.subcore_barrier()
        pltpu.sync_copy(acc.at[pl.ds(s * rps, rps)], out.at[c, pl.ds(s * rps, rps)])

    out_type = [jax.ShapeDtypeStruct((2, ACC_G2, WROW), _f32)]
    if grab_ty:
        out_type.append(jax.ShapeDtypeStruct((NLEFT, WROW), _f32))
    scratch = [
        pltpu.VMEM((NBR, CKR), jnp.int32),
        pltpu.VMEM((NBR, CKR), jnp.int32),
        pltpu.VMEM((NBR, CKR), jnp.int32),
        pltpu.VMEM((NBR, CKR), jnp.int32),
        pltpu.VMEM((NBR, CKR, WROW), _f32),
        pltpu.VMEM_SHARED((ACC_G2, WROW), _f32),
    ]
    if grab_ty:
        scratch += [pltpu.VMEM((NLEFT,), jnp.int32),
                    pltpu.VMEM((NLEFT, WROW), _f32)]
    scratch += [pltpu.SemaphoreType.DMA] * NBR
    return pl.kernel(body, out_type=tuple(out_type), mesh=_sc_mesh(),
                     scratch_types=scratch)


def _make_deg_scatter():
    rps = ACC_G2 // 16

    def body(dst, ones, zeros, out, didx, ones_v, acc, sem0, sem1):
        c = lax.axis_index("c")
        s = lax.axis_index("s")
        wid = s * 2 + c
        sems = (sem0, sem1)
        pltpu.sync_copy(zeros.at[pl.ds(s * rps, rps)], acc.at[pl.ds(s * rps, rps)])
        pltpu.sync_copy(ones, ones_v)

        def start(i, b):
            base = (wid * NC2 + i) * CK
            pltpu.async_copy(dst.at[pl.ds(base, CK)], didx.at[b], sems[b])

        start(0, 0)
        plsc.subcore_barrier()

        def pair(k, carry):
            i0 = k * 2
            for b in range(2):
                i = i0 + b
                nb = 1 - b

                @pl.when(i + 1 < NC2)
                def _():
                    start(i + 1, nb)

                base = (wid * NC2 + i) * CK
                pltpu.make_async_copy(dst.at[pl.ds(base, CK)], didx.at[b],
                                      sems[b]).wait()
                pltpu.sync_copy(ones_v, acc.at[didx.at[b]], add=True)
            return carry

        lax.fori_loop(0, NC2 // 2, pair, 0)
        plsc.subcore_barrier()
        pltpu.sync_copy(acc.at[pl.ds(s * rps, rps)], out.at[c, pl.ds(s * rps, rps)])

    return pl.kernel(
        body,
        out_type=jax.ShapeDtypeStruct((2, ACC_G2, WROW), _f32),
        mesh=_sc_mesh(),
        scratch_types=[
            pltpu.VMEM((2, CK), jnp.int32),
            pltpu.VMEM((CK, WROW), _f32),
            pltpu.VMEM_SHARED((ACC_G2, WROW), _f32),
            pltpu.SemaphoreType.DMA,
            pltpu.SemaphoreType.DMA,
        ],
    )


_BN = 400


def _kx1_body(acc_ref, x_ref):
    a = acc_ref[0] + acc_ref[1]
    deg = jnp.maximum(a[:, D0:D0 + 1], 1.0)
    x = jnp.maximum(a[:, :D0] / deg, 0.0)
    x_ref[...] = jnp.concatenate(
        [x, jnp.zeros((x.shape[0], WROW - D0), _f32)], axis=1)


def _kt_body(din, dout, x_ref, coeff_ref, bases_ref, out_ref):
    w = bases_ref[0] * coeff_ref[0, 0, 0]
    for b in range(1, 4):
        w = w + bases_ref[b] * coeff_ref[0, 0, b]
    xt = jnp.dot(x_ref[:, :din], w, preferred_element_type=_f32)
    if dout < WROW:
        xt = jnp.concatenate(
            [xt, jnp.zeros((xt.shape[0], WROW - dout), _f32)], axis=1)
    out_ref[...] = xt


def _kh_body(acc_ref, dg_ref, x_ref, ws_ref, b_ref, h_ref, deg_ref):
    a = acc_ref[0] + acc_ref[1]
    deg = jnp.maximum(dg_ref[0, :, 0:1] + dg_ref[1, :, 0:1], 1.0)
    deg_ref[...] = deg
    h = a / deg + jnp.dot(x_ref[:, :D0], ws_ref[...],
                          preferred_element_type=_f32) + b_ref[...]
    h_ref[...] = jnp.maximum(h, 0.0)


def _kf_body(acc_ref, deg_ref, h_ref, ws_ref, b_ref, ty_ref, w_ref, o_ref):
    a = acc_ref[0, :, :D2] + acc_ref[1, :, :D2]
    x = a / deg_ref[...] + jnp.dot(h_ref[...], ws_ref[...],
                                   preferred_element_type=_f32) + b_ref[...]
    vm = ty_ref[:, :D2] * w_ref[...]
    sc = jnp.dot(x, vm.T, preferred_element_type=_f32)
    m = jnp.max(sc, axis=1, keepdims=True)
    e = jnp.exp(sc - m)
    o_ref[...] = e / jnp.sum(e, axis=1, keepdims=True)


def _full(shape):
    return pl.BlockSpec(shape, lambda *_: tuple(0 for _ in shape))


def kernel(edge_index_g2, edge_type_g2, edge_index_g1, all_node_embedding,
           bases1, coeff1, w_self1, bias1, bases2, coeff2, w_self2, bias2,
           weights, left_common):
    i32 = jnp.int32
    src1 = jnp.concatenate([edge_index_g1[0].astype(i32),
                            jnp.zeros((E1P - E1,), i32)])
    dst1 = jnp.concatenate([edge_index_g1[1].astype(i32),
                            jnp.full((E1P - E1,), N1, i32)])
    src2 = jnp.concatenate([edge_index_g2[0].astype(i32),
                            jnp.zeros((E2P - E2,), i32)])
    et2 = jnp.concatenate([edge_type_g2.astype(i32),
                           jnp.zeros((E2P - E2,), i32)])
    dst2 = jnp.concatenate([edge_index_g2[1].astype(i32),
                            jnp.full((E2P - E2,), N2, i32)])
    xpad = jnp.concatenate(
        [all_node_embedding, jnp.ones((N1, 1), _f32),
         jnp.zeros((N1, WROW - D0 - 1), _f32)], axis=1)

    acc_g1 = _make_concept_scatter()(
        xpad, src1, dst1, jnp.zeros((ACC_G1, WROW), _f32))

    x_g1 = pl.pallas_call(
        _kx1_body,
        grid=(N1 // _BN,),
        in_specs=[pl.BlockSpec((2, _BN, WROW), lambda i: (0, i, 0))],
        out_specs=pl.BlockSpec((_BN, WROW), lambda i: (i, 0)),
        out_shape=jax.ShapeDtypeStruct((N1, WROW), _f32),
    )(acc_g1)

    nj = N2 // _BN
    table1 = pl.pallas_call(
        functools.partial(_kt_body, D0, D1),
        grid=(8, nj),
        in_specs=[
            pl.BlockSpec((_BN, WROW), lambda r, j: (j, 0)),
            pl.BlockSpec((1, 1, 4), lambda r, j: (r, 0, 0)),
            pl.BlockSpec((4, D0, D1), lambda r, j: (0, 0, 0)),
        ],
        out_specs=pl.BlockSpec((_BN, WROW), lambda r, j: (r * nj + j, 0)),
        out_shape=jax.ShapeDtypeStruct((8 * N2, WROW), _f32),
    )(x_g1, coeff1.reshape(8, 1, 4), bases1)

    acc1 = _make_rgcn_scatter(False)(
        table1, src2, et2, dst2, jnp.zeros((ACC_G2, WROW), _f32))[0]
    degp = _make_deg_scatter()(
        dst2, jnp.ones((CK, WROW), _f32), jnp.zeros((ACC_G2, WROW), _f32))

    h, deg2 = pl.pallas_call(
        _kh_body,
        grid=(nj,),
        in_specs=[
            pl.BlockSpec((2, _BN, WROW), lambda j: (0, j, 0)),
            pl.BlockSpec((2, _BN, WROW), lambda j: (0, j, 0)),
            pl.BlockSpec((_BN, WROW), lambda j: (j, 0)),
            _full((D0, D1)),
            _full((1, D1)),
        ],
        out_specs=[pl.BlockSpec((_BN, D1), lambda j: (j, 0)),
                   pl.BlockSpec((_BN, 1), lambda j: (j, 0))],
        out_shape=[jax.ShapeDtypeStruct((N2, D1), _f32),
                   jax.ShapeDtypeStruct((N2, 1), _f32)],
    )(acc1, degp, x_g1, w_self1, bias1.reshape(1, D1))

    table2 = pl.pallas_call(
        functools.partial(_kt_body, D1, D2),
        grid=(8, nj),
        in_specs=[
            pl.BlockSpec((_BN, D1), lambda r, j: (j, 0)),
            pl.BlockSpec((1, 1, 4), lambda r, j: (r, 0, 0)),
            pl.BlockSpec((4, D1, D2), lambda r, j: (0, 0, 0)),
        ],
        out_specs=pl.BlockSpec((_BN, WROW), lambda r, j: (r * nj + j, 0)),
        out_shape=jax.ShapeDtypeStruct((8 * N2, WROW), _f32),
    )(h, coeff2.reshape(8, 1, 4), bases2)

    acc2, ty = _make_rgcn_scatter(True)(
        table2, src2, et2, dst2,
        jnp.zeros((ACC_G2, WROW), _f32),
        x_g1, left_common.astype(i32))

    out = pl.pallas_call(
        _kf_body,
        grid=(nj,),
        in_specs=[
            pl.BlockSpec((2, _BN, WROW), lambda j: (0, j, 0)),
            pl.BlockSpec((_BN, 1), lambda j: (j, 0)),
            pl.BlockSpec((_BN, D1), lambda j: (j, 0)),
            _full((D1, D2)),
            _full((1, D2)),
            _full((NLEFT, WROW)),
            _full((1, D2)),
        ],
        out_specs=pl.BlockSpec((_BN, NLEFT), lambda j: (j, 0)),
        out_shape=jax.ShapeDtypeStruct((N2, NLEFT), _f32),
    )(acc2, deg2, h, w_self2, bias2.reshape(1, D2), ty,
      weights.reshape(1, D2))
    return out


# speedup vs baseline: 1.0133x; 1.0133x over previous
import functools

import jax
import jax.numpy as jnp
from jax import lax
from jax.experimental import pallas as pl
from jax.experimental.pallas import tpu as pltpu
from jax.experimental.pallas import tpu_sc as plsc

N1 = 12000
N2 = 10000
E1 = 384000
E2 = 320000
D0 = 64
D1 = 128
D2 = 64
NLEFT = 64

CK = 128
NTILES = 32
E1P = 385024
E2P = 327680
NC1 = E1P // (NTILES * CK)
NC2 = E2P // (NTILES * CK)
ACC_G1 = 12032
ACC_G2 = 10112
WROW = 128

_f32 = jnp.float32


def _sc_mesh():
    return plsc.VectorSubcoreMesh(core_axis_name="c", subcore_axis_name="s")


def _make_concept_scatter():
    rps = ACC_G1 // 16

    def body(tbl, src, dst, zeros, out, sidx, didx, rows, acc, sem0, sem1):
        c = lax.axis_index("c")
        s = lax.axis_index("s")
        wid = s * 2 + c
        sems = (sem0, sem1)
        pltpu.sync_copy(zeros.at[pl.ds(s * rps, rps)], acc.at[pl.ds(s * rps, rps)])

        def load(i, b):
            base = (wid * NC1 + i) * CK
            pltpu.sync_copy(src.at[pl.ds(base, CK)], sidx.at[b])
            pltpu.sync_copy(dst.at[pl.ds(base, CK)], didx.at[b])

        load(0, 0)
        pltpu.async_copy(tbl.at[sidx.at[0]], rows.at[0], sems[0])
        plsc.subcore_barrier()

        def pair(k, carry):
            i0 = k * 2
            for b in range(2):
                i = i0 + b
                nb = 1 - b

                @pl.when(i + 1 < NC1)
                def _():
                    load(i + 1, nb)
                    pltpu.async_copy(tbl.at[sidx.at[nb]], rows.at[nb], sems[nb])

                pltpu.make_async_copy(tbl.at[sidx.at[b]], rows.at[b],
                                      sems[b]).wait()
                pltpu.sync_copy(rows.at[b], acc.at[didx.at[b]], add=True)
            return carry

        lax.fori_loop(0, NC1 // 2, pair, 0)
        plsc.subcore_barrier()
        pltpu.sync_copy(acc.at[pl.ds(s * rps, rps)], out.at[c, pl.ds(s * rps, rps)])

    return pl.kernel(
        body,
        out_type=jax.ShapeDtypeStruct((2, ACC_G1, WROW), _f32),
        mesh=_sc_mesh(),
        scratch_types=[
            pltpu.VMEM((2, CK), jnp.int32),
            pltpu.VMEM((2, CK), jnp.int32),
            pltpu.VMEM((2, CK, WROW), _f32),
            pltpu.VMEM_SHARED((ACC_G1, WROW), _f32),
            pltpu.SemaphoreType.DMA,
            pltpu.SemaphoreType.DMA,
        ],
    )


def _make_rgcn_scatter(grab_ty):
    rps = ACC_G2 // 16

    def body(*refs):
        it = iter(refs)
        tbl = next(it); src = next(it); et = next(it); dst = next(it)
        zeros = next(it)
        if grab_ty:
            xg1 = next(it); lc = next(it)
        out = next(it)
        if grab_ty:
            ty_out = next(it)
        sidx = next(it); eidx = next(it); gidx = next(it); didx = next(it)
        rows = next(it)
        acc = next(it)
        if grab_ty:
            lidx = next(it); lrows = next(it)
        sem0 = next(it); sem1 = next(it)
        sems = (sem0, sem1)

        c = lax.axis_index("c")
        s = lax.axis_index("s")
        wid = s * 2 + c
        pltpu.sync_copy(zeros.at[pl.ds(s * rps, rps)], acc.at[pl.ds(s * rps, rps)])
        if grab_ty:
            @pl.when(wid == 0)
            def _():
                pltpu.sync_copy(lc, lidx)
                pltpu.async_copy(xg1.at[lidx], lrows, sems[0]).wait()
                pltpu.sync_copy(lrows, ty_out)

        def load(i, b):
            base = (wid * NC2 + i) * CK
            pltpu.sync_copy(src.at[pl.ds(base, CK)], sidx.at[b])
            pltpu.sync_copy(et.at[pl.ds(base, CK)], eidx.at[b])
            pltpu.sync_copy(dst.at[pl.ds(base, CK)], didx.at[b])
            for j in range(CK // 16):
                sl = pl.ds(j * 16, 16)
                gidx[b, sl] = eidx[b, sl] * N2 + sidx[b, sl]

        load(0, 0)
        pltpu.async_copy(tbl.at[gidx.at[0]], rows.at[0], sems[0])
        plsc.subcore_barrier()

        def pair(k, carry):
            i0 = k * 2
            for b in range(2):
                i = i0 + b
                nb = 1 - b

                @pl.when(i + 1 < NC2)
                def _():
                    load(i + 1, nb)
                    pltpu.async_copy(tbl.at[gidx.at[nb]], rows.at[nb], sems[nb])

                pltpu.make_async_copy(tbl.at[gidx.at[b]], rows.at[b],
                                      sems[b]).wait()
                pltpu.sync_copy(rows.at[b], acc.at[didx.at[b]], add=True)
            return carry

        lax.fori_loop(0, NC2 // 2, pair, 0)
        plsc.subcore_barrier()
        pltpu.sync_copy(acc.at[pl.ds(s * rps, rps)], out.at[c, pl.ds(s * rps, rps)])

    out_type = [jax.ShapeDtypeStruct((2, ACC_G2, WROW), _f32)]
    if grab_ty:
        out_type.append(jax.ShapeDtypeStruct((NLEFT, WROW), _f32))
    scratch = [
        pltpu.VMEM((2, CK), jnp.int32),
        pltpu.VMEM((2, CK), jnp.int32),
        pltpu.VMEM((2, CK), jnp.int32),
        pltpu.VMEM((2, CK), jnp.int32),
        pltpu.VMEM((2, CK, WROW), _f32),
        pltpu.VMEM_SHARED((ACC_G2, WROW), _f32),
    ]
    if grab_ty:
        scratch += [pltpu.VMEM((NLEFT,), jnp.int32),
                    pltpu.VMEM((NLEFT, WROW), _f32)]
    scratch += [pltpu.SemaphoreType.DMA, pltpu.SemaphoreType.DMA]
    return pl.kernel(body, out_type=tuple(out_type), mesh=_sc_mesh(),
                     scratch_types=scratch)


def _make_deg_scatter():
    rps = ACC_G2 // 16

    def body(dst, ones, zeros, out, didx, ones_v, acc, sem0, sem1):
        c = lax.axis_index("c")
        s = lax.axis_index("s")
        wid = s * 2 + c
        sems = (sem0, sem1)
        pltpu.sync_copy(zeros.at[pl.ds(s * rps, rps)], acc.at[pl.ds(s * rps, rps)])
        pltpu.sync_copy(ones, ones_v)

        def start(i, b):
            base = (wid * NC2 + i) * CK
            pltpu.async_copy(dst.at[pl.ds(base, CK)], didx.at[b], sems[b])

        start(0, 0)
        plsc.subcore_barrier()

        def pair(k, carry):
            i0 = k * 2
            for b in range(2):
                i = i0 + b
                nb = 1 - b

                @pl.when(i + 1 < NC2)
                def _():
                    start(i + 1, nb)

                base = (wid * NC2 + i) * CK
                pltpu.make_async_copy(dst.at[pl.ds(base, CK)], didx.at[b],
                                      sems[b]).wait()
                pltpu.sync_copy(ones_v, acc.at[didx.at[b]], add=True)
            return carry

        lax.fori_loop(0, NC2 // 2, pair, 0)
        plsc.subcore_barrier()
        pltpu.sync_copy(acc.at[pl.ds(s * rps, rps)], out.at[c, pl.ds(s * rps, rps)])

    return pl.kernel(
        body,
        out_type=jax.ShapeDtypeStruct((2, ACC_G2, WROW), _f32),
        mesh=_sc_mesh(),
        scratch_types=[
            pltpu.VMEM((2, CK), jnp.int32),
            pltpu.VMEM((CK, WROW), _f32),
            pltpu.VMEM_SHARED((ACC_G2, WROW), _f32),
            pltpu.SemaphoreType.DMA,
            pltpu.SemaphoreType.DMA,
        ],
    )


_BN = 400


def _kx1_body(acc_ref, x_ref):
    a = acc_ref[0] + acc_ref[1]
    deg = jnp.maximum(a[:, D0:D0 + 1], 1.0)
    x = jnp.maximum(a[:, :D0] / deg, 0.0)
    x_ref[...] = jnp.concatenate(
        [x, jnp.zeros((x.shape[0], WROW - D0), _f32)], axis=1)


def _kt_body(din, dout, x_ref, coeff_ref, bases_ref, out_ref):
    w = bases_ref[0] * coeff_ref[0, 0, 0]
    for b in range(1, 4):
        w = w + bases_ref[b] * coeff_ref[0, 0, b]
    xt = jnp.dot(x_ref[:, :din], w, preferred_element_type=_f32)
    if dout < WROW:
        xt = jnp.concatenate(
            [xt, jnp.zeros((xt.shape[0], WROW - dout), _f32)], axis=1)
    out_ref[...] = xt


def _kh_body(acc_ref, dg_ref, x_ref, ws_ref, b_ref, h_ref, deg_ref):
    a = acc_ref[0] + acc_ref[1]
    deg = jnp.maximum(dg_ref[0, :, 0:1] + dg_ref[1, :, 0:1], 1.0)
    deg_ref[...] = deg
    h = a / deg + jnp.dot(x_ref[:, :D0], ws_ref[...],
                          preferred_element_type=_f32) + b_ref[...]
    h_ref[...] = jnp.maximum(h, 0.0)


def _kf_body(acc_ref, deg_ref, h_ref, ws_ref, b_ref, ty_ref, w_ref, o_ref):
    a = acc_ref[0, :, :D2] + acc_ref[1, :, :D2]
    x = a / deg_ref[...] + jnp.dot(h_ref[...], ws_ref[...],
                                   preferred_element_type=_f32) + b_ref[...]
    vm = ty_ref[:, :D2] * w_ref[...]
    sc = jnp.dot(x, vm.T, preferred_element_type=_f32)
    m = jnp.max(sc, axis=1, keepdims=True)
    e = jnp.exp(sc - m)
    o_ref[...] = e / jnp.sum(e, axis=1, keepdims=True)


def _full(shape):
    return pl.BlockSpec(shape, lambda *_: tuple(0 for _ in shape))


def kernel(edge_index_g2, edge_type_g2, edge_index_g1, all_node_embedding,
           bases1, coeff1, w_self1, bias1, bases2, coeff2, w_self2, bias2,
           weights, left_common):
    i32 = jnp.int32
    src1 = jnp.concatenate([edge_index_g1[0].astype(i32),
                            jnp.zeros((E1P - E1,), i32)])
    dst1 = jnp.concatenate([edge_index_g1[1].astype(i32),
                            jnp.full((E1P - E1,), N1, i32)])
    src2 = jnp.concatenate([edge_index_g2[0].astype(i32),
                            jnp.zeros((E2P - E2,), i32)])
    et2 = jnp.concatenate([edge_type_g2.astype(i32),
                           jnp.zeros((E2P - E2,), i32)])
    dst2 = jnp.concatenate([edge_index_g2[1].astype(i32),
                            jnp.full((E2P - E2,), N2, i32)])
    xpad = jnp.concatenate(
        [all_node_embedding, jnp.ones((N1, 1), _f32),
         jnp.zeros((N1, WROW - D0 - 1), _f32)], axis=1)

    acc_g1 = _make_concept_scatter()(
        xpad, src1, dst1, jnp.zeros((ACC_G1, WROW), _f32))

    degp = _make_deg_scatter()(
        dst2, jnp.ones((CK, WROW), _f32), jnp.zeros((ACC_G2, WROW), _f32))

    x_g1 = pl.pallas_call(
        _kx1_body,
        grid=(N1 // _BN,),
        in_specs=[pl.BlockSpec((2, _BN, WROW), lambda i: (0, i, 0))],
        out_specs=pl.BlockSpec((_BN, WROW), lambda i: (i, 0)),
        out_shape=jax.ShapeDtypeStruct((N1, WROW), _f32),
    )(acc_g1)

    nj = N2 // _BN
    table1 = pl.pallas_call(
        functools.partial(_kt_body, D0, D1),
        grid=(8, nj),
        in_specs=[
            pl.BlockSpec((_BN, WROW), lambda r, j: (j, 0)),
            pl.BlockSpec((1, 1, 4), lambda r, j: (r, 0, 0)),
            pl.BlockSpec((4, D0, D1), lambda r, j: (0, 0, 0)),
        ],
        out_specs=pl.BlockSpec((_BN, WROW), lambda r, j: (r * nj + j, 0)),
        out_shape=jax.ShapeDtypeStruct((8 * N2, WROW), _f32),
    )(x_g1, coeff1.reshape(8, 1, 4), bases1)

    acc1 = _make_rgcn_scatter(False)(
        table1, src2, et2, dst2, jnp.zeros((ACC_G2, WROW), _f32))[0]

    h, deg2 = pl.pallas_call(
        _kh_body,
        grid=(nj,),
        in_specs=[
            pl.BlockSpec((2, _BN, WROW), lambda j: (0, j, 0)),
            pl.BlockSpec((2, _BN, WROW), lambda j: (0, j, 0)),
            pl.BlockSpec((_BN, WROW), lambda j: (j, 0)),
            _full((D0, D1)),
            _full((1, D1)),
        ],
        out_specs=[pl.BlockSpec((_BN, D1), lambda j: (j, 0)),
                   pl.BlockSpec((_BN, 1), lambda j: (j, 0))],
        out_shape=[jax.ShapeDtypeStruct((N2, D1), _f32),
                   jax.ShapeDtypeStruct((N2, 1), _f32)],
    )(acc1, degp, x_g1, w_self1, bias1.reshape(1, D1))

    table2 = pl.pallas_call(
        functools.partial(_kt_body, D1, D2),
        grid=(8, nj),
        in_specs=[
            pl.BlockSpec((_BN, D1), lambda r, j: (j, 0)),
            pl.BlockSpec((1, 1, 4), lambda r, j: (r, 0, 0)),
            pl.BlockSpec((4, D1, D2), lambda r, j: (0, 0, 0)),
        ],
        out_specs=pl.BlockSpec((_BN, WROW), lambda r, j: (r * nj + j, 0)),
        out_shape=jax.ShapeDtypeStruct((8 * N2, WROW), _f32),
    )(h, coeff2.reshape(8, 1, 4), bases2)

    acc2, ty = _make_rgcn_scatter(True)(
        table2, src2, et2, dst2,
        jnp.zeros((ACC_G2, WROW), _f32),
        x_g1, left_common.astype(i32))

    out = pl.pallas_call(
        _kf_body,
        grid=(nj,),
        in_specs=[
            pl.BlockSpec((2, _BN, WROW), lambda j: (0, j, 0)),
            pl.BlockSpec((_BN, 1), lambda j: (j, 0)),
            pl.BlockSpec((_BN, D1), lambda j: (j, 0)),
            _full((D1, D2)),
            _full((1, D2)),
            _full((NLEFT, WROW)),
            _full((1, D2)),
        ],
        out_specs=pl.BlockSpec((_BN, NLEFT), lambda j: (j, 0)),
        out_shape=jax.ShapeDtypeStruct((N2, NLEFT), _f32),
    )(acc2, deg2, h, w_self2, bias2.reshape(1, D2), ty,
      weights.reshape(1, D2))
    return out


# speedup vs baseline: 1.0360x; 1.0224x over previous
import functools

import jax
import jax.numpy as jnp
from jax import lax
from jax.experimental import pallas as pl
from jax.experimental.pallas import tpu as pltpu
from jax.experimental.pallas import tpu_sc as plsc

N1 = 12000
N2 = 10000
E1 = 384000
E2 = 320000
D0 = 64
D1 = 128
D2 = 64
NLEFT = 64

CK = 128
NTILES = 32
E1P = 385024
E2P = 327680
NC1 = E1P // (NTILES * CK)
NC2 = E2P // (NTILES * CK)
ACC_G1 = 12032
ACC_G2 = 10112
WROW = 128

_f32 = jnp.float32


def _sc_mesh():
    return plsc.VectorSubcoreMesh(core_axis_name="c", subcore_axis_name="s")


def _make_concept_scatter():
    rps = ACC_G1 // 16

    def body(tbl, src, dst, zeros, out, sidx, didx, rows, acc, sem0, sem1):
        c = lax.axis_index("c")
        s = lax.axis_index("s")
        wid = s * 2 + c
        sems = (sem0, sem1)
        pltpu.sync_copy(zeros.at[pl.ds(s * rps, rps)], acc.at[pl.ds(s * rps, rps)])

        def load(i, b):
            base = (wid * NC1 + i) * CK
            pltpu.sync_copy(src.at[pl.ds(base, CK)], sidx.at[b])
            pltpu.sync_copy(dst.at[pl.ds(base, CK)], didx.at[b])

        load(0, 0)
        pltpu.async_copy(tbl.at[sidx.at[0]], rows.at[0], sems[0])
        plsc.subcore_barrier()

        def pair(k, carry):
            i0 = k * 2
            for b in range(2):
                i = i0 + b
                nb = 1 - b

                @pl.when(i + 1 < NC1)
                def _():
                    load(i + 1, nb)
                    pltpu.async_copy(tbl.at[sidx.at[nb]], rows.at[nb], sems[nb])

                pltpu.make_async_copy(tbl.at[sidx.at[b]], rows.at[b],
                                      sems[b]).wait()
                pltpu.sync_copy(rows.at[b], acc.at[didx.at[b]], add=True)
            return carry

        lax.fori_loop(0, NC1 // 2, pair, 0)
        plsc.subcore_barrier()
        pltpu.sync_copy(acc.at[pl.ds(s * rps, rps)], out.at[c, pl.ds(s * rps, rps)])

    return pl.kernel(
        body,
        out_type=jax.ShapeDtypeStruct((2, ACC_G1, WROW), _f32),
        mesh=_sc_mesh(),
        scratch_types=[
            pltpu.VMEM((2, CK), jnp.int32),
            pltpu.VMEM((2, CK), jnp.int32),
            pltpu.VMEM((2, CK, WROW), _f32),
            pltpu.VMEM_SHARED((ACC_G1, WROW), _f32),
            pltpu.SemaphoreType.DMA,
            pltpu.SemaphoreType.DMA,
        ],
    )


def _make_rgcn_scatter(grab_ty, tstride):
    rps = ACC_G2 // 16

    def body(*refs):
        it = iter(refs)
        tbl = next(it); src = next(it); et = next(it); dst = next(it)
        zeros = next(it)
        if grab_ty:
            xg1 = next(it); lc = next(it)
        out = next(it)
        if grab_ty:
            ty_out = next(it)
        sidx = next(it); eidx = next(it); gidx = next(it); didx = next(it)
        rows = next(it)
        acc = next(it)
        if grab_ty:
            lidx = next(it); lrows = next(it)
        sem0 = next(it); sem1 = next(it)
        sems = (sem0, sem1)

        c = lax.axis_index("c")
        s = lax.axis_index("s")
        wid = s * 2 + c
        pltpu.sync_copy(zeros.at[pl.ds(s * rps, rps)], acc.at[pl.ds(s * rps, rps)])
        if grab_ty:
            @pl.when(wid == 0)
            def _():
                pltpu.sync_copy(lc, lidx)
                pltpu.async_copy(xg1.at[lidx], lrows, sems[0]).wait()
                pltpu.sync_copy(lrows, ty_out)

        def load(i, b):
            base = (wid * NC2 + i) * CK
            pltpu.sync_copy(src.at[pl.ds(base, CK)], sidx.at[b])
            pltpu.sync_copy(et.at[pl.ds(base, CK)], eidx.at[b])
            pltpu.sync_copy(dst.at[pl.ds(base, CK)], didx.at[b])
            for j in range(CK // 16):
                sl = pl.ds(j * 16, 16)
                gidx[b, sl] = eidx[b, sl] * tstride + sidx[b, sl]

        load(0, 0)
        pltpu.async_copy(tbl.at[gidx.at[0]], rows.at[0], sems[0])
        plsc.subcore_barrier()

        def pair(k, carry):
            i0 = k * 2
            for b in range(2):
                i = i0 + b
                nb = 1 - b

                @pl.when(i + 1 < NC2)
                def _():
                    load(i + 1, nb)
                    pltpu.async_copy(tbl.at[gidx.at[nb]], rows.at[nb], sems[nb])

                pltpu.make_async_copy(tbl.at[gidx.at[b]], rows.at[b],
                                      sems[b]).wait()
                pltpu.sync_copy(rows.at[b], acc.at[didx.at[b]], add=True)
            return carry

        lax.fori_loop(0, NC2 // 2, pair, 0)
        plsc.subcore_barrier()
        pltpu.sync_copy(acc.at[pl.ds(s * rps, rps)], out.at[c, pl.ds(s * rps, rps)])

    out_type = [jax.ShapeDtypeStruct((2, ACC_G2, WROW), _f32)]
    if grab_ty:
        out_type.append(jax.ShapeDtypeStruct((NLEFT, WROW), _f32))
    scratch = [
        pltpu.VMEM((2, CK), jnp.int32),
        pltpu.VMEM((2, CK), jnp.int32),
        pltpu.VMEM((2, CK), jnp.int32),
        pltpu.VMEM((2, CK), jnp.int32),
        pltpu.VMEM((2, CK, WROW), _f32),
        pltpu.VMEM_SHARED((ACC_G2, WROW), _f32),
    ]
    if grab_ty:
        scratch += [pltpu.VMEM((NLEFT,), jnp.int32),
                    pltpu.VMEM((NLEFT, WROW), _f32)]
    scratch += [pltpu.SemaphoreType.DMA, pltpu.SemaphoreType.DMA]
    return pl.kernel(body, out_type=tuple(out_type), mesh=_sc_mesh(),
                     scratch_types=scratch)


def _make_deg_scatter():
    rps = ACC_G2 // 16

    def body(dst, ones, zeros, out, didx, ones_v, acc, sem0, sem1):
        c = lax.axis_index("c")
        s = lax.axis_index("s")
        wid = s * 2 + c
        sems = (sem0, sem1)
        pltpu.sync_copy(zeros.at[pl.ds(s * rps, rps)], acc.at[pl.ds(s * rps, rps)])
        pltpu.sync_copy(ones, ones_v)

        def start(i, b):
            base = (wid * NC2 + i) * CK
            pltpu.async_copy(dst.at[pl.ds(base, CK)], didx.at[b], sems[b])

        start(0, 0)
        plsc.subcore_barrier()

        def pair(k, carry):
            i0 = k * 2
            for b in range(2):
                i = i0 + b
                nb = 1 - b

                @pl.when(i + 1 < NC2)
                def _():
                    start(i + 1, nb)

                base = (wid * NC2 + i) * CK
                pltpu.make_async_copy(dst.at[pl.ds(base, CK)], didx.at[b],
                                      sems[b]).wait()
                pltpu.sync_copy(ones_v, acc.at[didx.at[b]], add=True)
            return carry

        lax.fori_loop(0, NC2 // 2, pair, 0)
        plsc.subcore_barrier()
        pltpu.sync_copy(acc.at[pl.ds(s * rps, rps)], out.at[c, pl.ds(s * rps, rps)])

    return pl.kernel(
        body,
        out_type=jax.ShapeDtypeStruct((2, ACC_G2, WROW), _f32),
        mesh=_sc_mesh(),
        scratch_types=[
            pltpu.VMEM((2, CK), jnp.int32),
            pltpu.VMEM((CK, WROW), _f32),
            pltpu.VMEM_SHARED((ACC_G2, WROW), _f32),
            pltpu.SemaphoreType.DMA,
            pltpu.SemaphoreType.DMA,
        ],
    )


_BN = 400


def _kt1x_body(acc_ref, coeff_ref, bases_ref, tbl_ref, x_ref):
    a = acc_ref[0] + acc_ref[1]
    deg = jnp.maximum(a[:, D0:D0 + 1], 1.0)
    x = jnp.maximum(a[:, :D0] / deg, 0.0)
    x_ref[...] = jnp.concatenate(
        [x, jnp.zeros((x.shape[0], WROW - D0), _f32)], axis=1)
    w = bases_ref[0] * coeff_ref[0, 0, 0]
    for b in range(1, 4):
        w = w + bases_ref[b] * coeff_ref[0, 0, b]
    tbl_ref[...] = jnp.dot(x, w, preferred_element_type=_f32)


def _kht2_body(acc_ref, dg_ref, x_ref, ws_ref, b_ref, coeff_ref, bases_ref,
               tbl_ref, h_ref, deg_ref):
    a = acc_ref[0] + acc_ref[1]
    deg = jnp.maximum(dg_ref[0, :, 0:1] + dg_ref[1, :, 0:1], 1.0)
    deg_ref[...] = deg
    h = jnp.maximum(a / deg + jnp.dot(x_ref[:, :D0], ws_ref[...],
                                      preferred_element_type=_f32) + b_ref[...],
                    0.0)
    h_ref[...] = h
    w = bases_ref[0] * coeff_ref[0, 0, 0]
    for b in range(1, 4):
        w = w + bases_ref[b] * coeff_ref[0, 0, b]
    xt = jnp.dot(h, w, preferred_element_type=_f32)
    tbl_ref[...] = jnp.concatenate(
        [xt, jnp.zeros((xt.shape[0], WROW - D2), _f32)], axis=1)


def _kt_body(din, dout, x_ref, coeff_ref, bases_ref, out_ref):
    w = bases_ref[0] * coeff_ref[0, 0, 0]
    for b in range(1, 4):
        w = w + bases_ref[b] * coeff_ref[0, 0, b]
    xt = jnp.dot(x_ref[:, :din], w, preferred_element_type=_f32)
    if dout < WROW:
        xt = jnp.concatenate(
            [xt, jnp.zeros((xt.shape[0], WROW - dout), _f32)], axis=1)
    out_ref[...] = xt


def _kh_body(acc_ref, dg_ref, x_ref, ws_ref, b_ref, h_ref, deg_ref):
    a = acc_ref[0] + acc_ref[1]
    deg = jnp.maximum(dg_ref[0, :, 0:1] + dg_ref[1, :, 0:1], 1.0)
    deg_ref[...] = deg
    h = a / deg + jnp.dot(x_ref[:, :D0], ws_ref[...],
                          preferred_element_type=_f32) + b_ref[...]
    h_ref[...] = jnp.maximum(h, 0.0)


def _kf_body(acc_ref, deg_ref, h_ref, ws_ref, b_ref, ty_ref, w_ref, o_ref):
    a = acc_ref[0, :, :D2] + acc_ref[1, :, :D2]
    x = a / deg_ref[...] + jnp.dot(h_ref[...], ws_ref[...],
                                   preferred_element_type=_f32) + b_ref[...]
    vm = ty_ref[:, :D2] * w_ref[...]
    sc = jnp.dot(x, vm.T, preferred_element_type=_f32)
    m = jnp.max(sc, axis=1, keepdims=True)
    e = jnp.exp(sc - m)
    o_ref[...] = e / jnp.sum(e, axis=1, keepdims=True)


def _full(shape):
    return pl.BlockSpec(shape, lambda *_: tuple(0 for _ in shape))


def kernel(edge_index_g2, edge_type_g2, edge_index_g1, all_node_embedding,
           bases1, coeff1, w_self1, bias1, bases2, coeff2, w_self2, bias2,
           weights, left_common):
    i32 = jnp.int32
    src1 = jnp.concatenate([edge_index_g1[0].astype(i32),
                            jnp.zeros((E1P - E1,), i32)])
    dst1 = jnp.concatenate([edge_index_g1[1].astype(i32),
                            jnp.full((E1P - E1,), N1, i32)])
    src2 = jnp.concatenate([edge_index_g2[0].astype(i32),
                            jnp.zeros((E2P - E2,), i32)])
    et2 = jnp.concatenate([edge_type_g2.astype(i32),
                           jnp.zeros((E2P - E2,), i32)])
    dst2 = jnp.concatenate([edge_index_g2[1].astype(i32),
                            jnp.full((E2P - E2,), N2, i32)])
    xpad = jnp.concatenate(
        [all_node_embedding, jnp.ones((N1, 1), _f32),
         jnp.zeros((N1, WROW - D0 - 1), _f32)], axis=1)

    acc_g1 = _make_concept_scatter()(
        xpad, src1, dst1, jnp.zeros((ACC_G1, WROW), _f32))

    degp = _make_deg_scatter()(
        dst2, jnp.ones((CK, WROW), _f32), jnp.zeros((ACC_G2, WROW), _f32))

    nj = N2 // _BN
    nj1 = N1 // _BN
    table1, x_g1 = pl.pallas_call(
        _kt1x_body,
        grid=(8, nj1),
        in_specs=[
            pl.BlockSpec((2, _BN, WROW), lambda r, j: (0, j, 0)),
            pl.BlockSpec((1, 1, 4), lambda r, j: (r, 0, 0)),
            pl.BlockSpec((4, D0, D1), lambda r, j: (0, 0, 0)),
        ],
        out_specs=[pl.BlockSpec((_BN, WROW), lambda r, j: (r * nj1 + j, 0)),
                   pl.BlockSpec((_BN, WROW), lambda r, j: (j, 0))],
        out_shape=[jax.ShapeDtypeStruct((8 * N1, WROW), _f32),
                   jax.ShapeDtypeStruct((N1, WROW), _f32)],
    )(acc_g1, coeff1.reshape(8, 1, 4), bases1)

    acc1 = _make_rgcn_scatter(False, N1)(
        table1, src2, et2, dst2, jnp.zeros((ACC_G2, WROW), _f32))[0]

    table2, h, deg2 = pl.pallas_call(
        _kht2_body,
        grid=(8, nj),
        in_specs=[
            pl.BlockSpec((2, _BN, WROW), lambda r, j: (0, j, 0)),
            pl.BlockSpec((2, _BN, WROW), lambda r, j: (0, j, 0)),
            pl.BlockSpec((_BN, WROW), lambda r, j: (j, 0)),
            _full((D0, D1)),
            _full((1, D1)),
            pl.BlockSpec((1, 1, 4), lambda r, j: (r, 0, 0)),
            _full((4, D1, D2)),
        ],
        out_specs=[pl.BlockSpec((_BN, WROW), lambda r, j: (r * nj + j, 0)),
                   pl.BlockSpec((_BN, D1), lambda r, j: (j, 0)),
                   pl.BlockSpec((_BN, 1), lambda r, j: (j, 0))],
        out_shape=[jax.ShapeDtypeStruct((8 * N2, WROW), _f32),
                   jax.ShapeDtypeStruct((N2, D1), _f32),
                   jax.ShapeDtypeStruct((N2, 1), _f32)],
    )(acc1, degp, x_g1, w_self1, bias1.reshape(1, D1),
      coeff2.reshape(8, 1, 4), bases2)

    acc2, ty = _make_rgcn_scatter(True, N2)(
        table2, src2, et2, dst2,
        jnp.zeros((ACC_G2, WROW), _f32),
        x_g1, left_common.astype(i32))

    out = pl.pallas_call(
        _kf_body,
        grid=(nj,),
        in_specs=[
            pl.BlockSpec((2, _BN, WROW), lambda j: (0, j, 0)),
            pl.BlockSpec((_BN, 1), lambda j: (j, 0)),
            pl.BlockSpec((_BN, D1), lambda j: (j, 0)),
            _full((D1, D2)),
            _full((1, D2)),
            _full((NLEFT, WROW)),
            _full((1, D2)),
        ],
        out_specs=pl.BlockSpec((_BN, NLEFT), lambda j: (j, 0)),
        out_shape=jax.ShapeDtypeStruct((N2, NLEFT), _f32),
    )(acc2, deg2, h, w_self2, bias2.reshape(1, D2), ty,
      weights.reshape(1, D2))
    return out
